# Initial kernel scaffold; baseline (speedup 1.0000x reference)
import jax
import jax.numpy as jnp
from jax.experimental import pallas as pl


def kernel(feat, edge_index, W1, b1, W2, b2):
    raise NotImplementedError("write your pallas kernel here")


# speedup vs baseline: 4.3160x; 4.3160x over previous
import functools

import jax
import jax.numpy as jnp
from jax import lax
from jax.experimental import pallas as pl
from jax.experimental.pallas import tpu as pltpu
from jax.experimental.pallas import tpu_sc as plsc

N = 10000
E = 320000
D = 128

NUM_CORES = 2
NUM_SUBCORES = 16
NW = NUM_CORES * NUM_SUBCORES

CH = 128
NP = 10240
ROWS_PER_TILE = NP // NUM_SUBCORES
EPAD = 323584
EPW = EPAD // NW
NCHUNK = EPW // CH
DUMMY_DST = N + 100

_sc_mesh = plsc.VectorSubcoreMesh(core_axis_name="c", subcore_axis_name="s")


@functools.partial(
    pl.kernel,
    out_type=jax.ShapeDtypeStruct((NUM_CORES, NP, D), jnp.float32),
    mesh=_sc_mesh,
    scratch_types=[
        pltpu.VMEM((CH,), jnp.int32),
        pltpu.VMEM((CH,), jnp.int32),
        pltpu.VMEM((CH, D), jnp.float32),
        pltpu.VMEM_SHARED((NP, D), jnp.float32),
        pltpu.SemaphoreType.DMA,
    ],
)
def _sc_segment_sum(table_hbm, src_hbm, dst_hbm, out_hbm,
                    sidx_v, didx_v, rows_v, acc_sh, sem):
    cid = lax.axis_index("c")
    sid = lax.axis_index("s")
    wid = cid * NUM_SUBCORES + sid
    base = wid * EPW

    @pl.loop(0, CH)
    def _zrow(r):
        @pl.loop(0, D, step=16)
        def _zcol(k):
            rows_v[r, pl.ds(k, 16)] = jnp.zeros((16,), jnp.float32)

    @pl.loop(0, ROWS_PER_TILE // CH)
    def _zacc(b):
        pltpu.sync_copy(rows_v, acc_sh.at[pl.ds(sid * ROWS_PER_TILE + b * CH, CH)])

    plsc.subcore_barrier()

    @pl.loop(0, NCHUNK)
    def _chunk(j):
        off = base + j * CH
        pltpu.sync_copy(src_hbm.at[pl.ds(off, CH)], sidx_v)
        pltpu.sync_copy(dst_hbm.at[pl.ds(off, CH)], didx_v)
        pltpu.async_copy(table_hbm.at[sidx_v], rows_v, sem).wait()
        pltpu.sync_copy(rows_v, acc_sh.at[didx_v], add=True)

    plsc.subcore_barrier()

    @pl.loop(0, ROWS_PER_TILE // CH)
    def _wout(b):
        r0 = sid * ROWS_PER_TILE + b * CH
        pltpu.sync_copy(acc_sh.at[pl.ds(r0, CH)], out_hbm.at[cid, pl.ds(r0, CH)])


def _dense_layer_body(p_ref, w_ref, b_ref, o_ref, *, activate):
    x = p_ref[0] + p_ref[1]
    y = jnp.dot(x, w_ref[...], preferred_element_type=jnp.float32) + b_ref[...]
    if activate:
        y = jnp.tanh(y)
    o_ref[...] = y


def _dense_layer(p, wt, b, activate):
    blk = 1024
    return pl.pallas_call(
        functools.partial(_dense_layer_body, activate=activate),
        grid=(NP // blk,),
        in_specs=[
            pl.BlockSpec((NUM_CORES, blk, D), lambda i: (0, i, 0)),
            pl.BlockSpec((D, D), lambda i: (0, 0)),
            pl.BlockSpec((1, D), lambda i: (0, 0)),
        ],
        out_specs=pl.BlockSpec((blk, D), lambda i: (i, 0)),
        out_shape=jax.ShapeDtypeStruct((NP, D), jnp.float32),
    )(p, wt, b)


@jax.jit
def kernel(feat, edge_index, W1, b1, W2, b2):
    src = edge_index[0]
    dst = edge_index[1]
    src_p = jnp.concatenate([src, jnp.zeros((EPAD - E,), jnp.int32)])
    dst_p = jnp.concatenate([dst, jnp.full((EPAD - E,), DUMMY_DST, jnp.int32)])
    feat_p = jnp.pad(feat, ((0, NP - N), (0, 0)))

    p1 = _sc_segment_sum(feat_p, src_p, dst_p)
    h = _dense_layer(p1, W1.T, b1.reshape(1, D), activate=True)
    p2 = _sc_segment_sum(h, src_p, dst_p)
    out = _dense_layer(p2, W2.T, b2.reshape(1, D), activate=False)
    return out[:N]


# speedup vs baseline: 9.9027x; 2.2944x over previous
import functools

import jax
import jax.numpy as jnp
from jax import lax
from jax.experimental import pallas as pl
from jax.experimental.pallas import tpu as pltpu
from jax.experimental.pallas import tpu_sc as plsc

N = 10000
E = 320000
D = 128

NUM_CORES = 2
NUM_SUBCORES = 16
NW = NUM_CORES * NUM_SUBCORES

CH = 128
B = 20
NBLK = 4
NCHUNK = B * NBLK
NP = 10240
ROWS_PER_TILE = NP // NUM_SUBCORES
EPW = NCHUNK * CH
EPAD = NW * EPW

_sc_mesh = plsc.VectorSubcoreMesh(core_axis_name="c", subcore_axis_name="s")


@functools.partial(
    pl.kernel,
    out_type=jax.ShapeDtypeStruct((NUM_CORES, NP, D), jnp.float32),
    mesh=_sc_mesh,
    scratch_types=[
        pltpu.VMEM((B, CH), jnp.int32),
        pltpu.VMEM((B, CH), jnp.int32),
        pltpu.VMEM((B, CH), jnp.int32),
        pltpu.VMEM((B, CH), jnp.int32),
        pltpu.VMEM((CH, D), jnp.float32),
        pltpu.VMEM((CH, D), jnp.float32),
        pltpu.VMEM_SHARED((NP, D), jnp.float32),
        pltpu.SemaphoreType.DMA,
        pltpu.SemaphoreType.DMA,
        pltpu.SemaphoreType.DMA,
        pltpu.SemaphoreType.DMA,
        pltpu.SemaphoreType.DMA,
        pltpu.SemaphoreType.DMA,
    ],
)
def _sc_segment_sum(table_hbm, src_hbm, dst_hbm, out_hbm,
                    sidx_a, sidx_b, didx_a, didx_b, rows0, rows1, acc_sh,
                    sem_g0, sem_g1, sem_s0, sem_s1, sem_ia, sem_ib):
    cid = lax.axis_index("c")
    sid = lax.axis_index("s")
    wid = cid * NUM_SUBCORES + sid

    sidx = [sidx_a, sidx_b]
    didx = [didx_a, didx_b]
    rows = [rows0, rows1]
    sem_g = [sem_g0, sem_g1]
    sem_s = [sem_s0, sem_s1]
    sem_i = [sem_ia, sem_ib]

    @pl.loop(0, CH)
    def _zrow(r):
        @pl.loop(0, D, step=16)
        def _zcol(k):
            rows0[r, pl.ds(k, 16)] = jnp.zeros((16,), jnp.float32)

    for r0 in range(0, ROWS_PER_TILE, CH):
        pltpu.sync_copy(rows0, acc_sh.at[pl.ds(sid * ROWS_PER_TILE + r0, CH)])

    plsc.subcore_barrier()

    def start_gather(idx_row, rb):
        pltpu.async_copy(table_hbm.at[idx_row], rows[rb], sem_g[rb])

    def wait_gather(idx_row, rb):
        pltpu.make_async_copy(table_hbm.at[idx_row], rows[rb], sem_g[rb]).wait()

    def start_scatter(idx_row, rb):
        pltpu.async_copy(rows[rb], acc_sh.at[idx_row], sem_s[rb], add=True)

    def wait_scatter(idx_row, rb):
        pltpu.make_async_copy(rows[rb], acc_sh.at[idx_row], sem_s[rb]).wait()

    pltpu.sync_copy(src_hbm.at[wid, 0], sidx[0])
    pltpu.sync_copy(dst_hbm.at[wid, 0], didx[0])
    start_gather(sidx[0].at[0], 0)
    start_gather(sidx[0].at[1], 1)

    for blk in range(NBLK):
        cur = blk % 2
        nxt = 1 - cur
        if blk + 1 < NBLK:
            pltpu.async_copy(src_hbm.at[wid, blk + 1], sidx[nxt], sem_i[nxt])
            pltpu.async_copy(dst_hbm.at[wid, blk + 1], didx[nxt], sem_i[nxt])
        for jj in range(0, B, 2):
            for u in range(2):
                wait_gather(sidx[cur].at[jj + u], u)
                start_scatter(didx[cur].at[jj + u], u)
            if jj + 2 < B:
                for u in range(2):
                    wait_scatter(didx[cur].at[jj + u], u)
                    start_gather(sidx[cur].at[jj + 2 + u], u)
            elif blk + 1 < NBLK:
                pltpu.make_async_copy(src_hbm.at[wid, blk + 1], sidx[nxt],
                                      sem_i[nxt]).wait()
                pltpu.make_async_copy(dst_hbm.at[wid, blk + 1], didx[nxt],
                                      sem_i[nxt]).wait()
                for u in range(2):
                    wait_scatter(didx[cur].at[jj + u], u)
                    start_gather(sidx[nxt].at[u], u)
            else:
                for u in range(2):
                    wait_scatter(didx[cur].at[jj + u], u)

    plsc.subcore_barrier()

    for r0 in range(0, ROWS_PER_TILE, CH):
        a0 = sid * ROWS_PER_TILE + r0
        pltpu.sync_copy(acc_sh.at[pl.ds(a0, CH)], out_hbm.at[cid, pl.ds(a0, CH)])


def _dense_layer_body(p_ref, w_ref, b_ref, o_ref, *, activate):
    x = p_ref[0] + p_ref[1]
    y = jnp.dot(x, w_ref[...], preferred_element_type=jnp.float32) + b_ref[...]
    if activate:
        y = jnp.tanh(y)
    o_ref[...] = y


def _dense_layer(p, wt, b, activate):
    blk = 1024
    return pl.pallas_call(
        functools.partial(_dense_layer_body, activate=activate),
        grid=(NP // blk,),
        in_specs=[
            pl.BlockSpec((NUM_CORES, blk, D), lambda i: (0, i, 0)),
            pl.BlockSpec((D, D), lambda i: (0, 0)),
            pl.BlockSpec((1, D), lambda i: (0, 0)),
        ],
        out_specs=pl.BlockSpec((blk, D), lambda i: (i, 0)),
        out_shape=jax.ShapeDtypeStruct((NP, D), jnp.float32),
    )(p, wt, b)


@jax.jit
def kernel(feat, edge_index, W1, b1, W2, b2):
    src = edge_index[0]
    dst = edge_index[1]
    npad = EPAD - E
    pad_src = jnp.arange(npad, dtype=jnp.int32) % N
    pad_dst = N + jnp.arange(npad, dtype=jnp.int32) % (NP - N)
    src4 = jnp.concatenate([src, pad_src]).reshape(NW, NBLK, B, CH)
    dst4 = jnp.concatenate([dst, pad_dst]).reshape(NW, NBLK, B, CH)
    feat_p = jnp.pad(feat, ((0, NP - N), (0, 0)))

    p1 = _sc_segment_sum(feat_p, src4, dst4)
    h = _dense_layer(p1, W1.T, b1.reshape(1, D), activate=True)
    p2 = _sc_segment_sum(h, src4, dst4)
    out = _dense_layer(p2, W2.T, b2.reshape(1, D), activate=False)
    return out[:N]


# speedup vs baseline: 11.8326x; 1.1949x over previous
import functools

import jax
import jax.numpy as jnp
from jax import lax
from jax.experimental import pallas as pl
from jax.experimental.pallas import tpu as pltpu
from jax.experimental.pallas import tpu_sc as plsc

N = 10000
E = 320000
D = 128

NUM_CORES = 2
NUM_SUBCORES = 16
NW = NUM_CORES * NUM_SUBCORES

CH = 64
NBUF = 4
B = 20
NBLK = 8
NCHUNK = B * NBLK
NP = 10240
ROWS_PER_TILE = NP // NUM_SUBCORES
EPW = NCHUNK * CH
EPAD = NW * EPW

_sc_mesh = plsc.VectorSubcoreMesh(core_axis_name="c", subcore_axis_name="s")


@functools.partial(
    pl.kernel,
    out_type=jax.ShapeDtypeStruct((NUM_CORES, NP, D), jnp.float32),
    mesh=_sc_mesh,
    scratch_types=[
        pltpu.VMEM((B, CH), jnp.int32),
        pltpu.VMEM((B, CH), jnp.int32),
        pltpu.VMEM((B, CH), jnp.int32),
        pltpu.VMEM((B, CH), jnp.int32),
        *[pltpu.VMEM((CH, D), jnp.float32) for _ in range(NBUF)],
        pltpu.VMEM_SHARED((NP, D), jnp.float32),
        *[pltpu.SemaphoreType.DMA for _ in range(NBUF)],
        *[pltpu.SemaphoreType.DMA for _ in range(NBUF)],
        pltpu.SemaphoreType.DMA,
        pltpu.SemaphoreType.DMA,
    ],
)
def _sc_segment_sum(table_hbm, src_hbm, dst_hbm, out_hbm,
                    sidx_a, sidx_b, didx_a, didx_b, *scr):
    rows = list(scr[:NBUF])
    acc_sh = scr[NBUF]
    sem_g = list(scr[NBUF + 1:2 * NBUF + 1])
    sem_s = list(scr[2 * NBUF + 1:3 * NBUF + 1])
    sem_ia, sem_ib = scr[3 * NBUF + 1], scr[3 * NBUF + 2]

    cid = lax.axis_index("c")
    sid = lax.axis_index("s")
    wid = cid * NUM_SUBCORES + sid

    sidx = [sidx_a, sidx_b]
    didx = [didx_a, didx_b]
    sem_i = [sem_ia, sem_ib]
    rows0 = rows[0]

    @pl.loop(0, CH)
    def _zrow(r):
        @pl.loop(0, D, step=16)
        def _zcol(k):
            rows0[r, pl.ds(k, 16)] = jnp.zeros((16,), jnp.float32)

    for r0 in range(0, ROWS_PER_TILE, CH):
        pltpu.sync_copy(rows0, acc_sh.at[pl.ds(sid * ROWS_PER_TILE + r0, CH)])

    plsc.subcore_barrier()

    def start_gather(idx_row, rb):
        pltpu.async_copy(table_hbm.at[idx_row], rows[rb], sem_g[rb])

    def wait_gather(idx_row, rb):
        pltpu.make_async_copy(table_hbm.at[idx_row], rows[rb], sem_g[rb]).wait()

    def start_scatter(idx_row, rb):
        pltpu.async_copy(rows[rb], acc_sh.at[idx_row], sem_s[rb], add=True)

    def wait_scatter(idx_row, rb):
        pltpu.make_async_copy(rows[rb], acc_sh.at[idx_row], sem_s[rb]).wait()

    pltpu.sync_copy(src_hbm.at[wid, 0], sidx[0])
    pltpu.sync_copy(dst_hbm.at[wid, 0], didx[0])
    for u in range(NBUF):
        start_gather(sidx[0].at[u], u)

    for blk in range(NBLK):
        cur = blk % 2
        nxt = 1 - cur
        if blk + 1 < NBLK:
            pltpu.async_copy(src_hbm.at[wid, blk + 1], sidx[nxt], sem_i[nxt])
            pltpu.async_copy(dst_hbm.at[wid, blk + 1], didx[nxt], sem_i[nxt])
        for jj in range(0, B, NBUF):
            for u in range(NBUF):
                wait_gather(sidx[cur].at[jj + u], u)
                start_scatter(didx[cur].at[jj + u], u)
            if jj + NBUF < B:
                for u in range(NBUF):
                    wait_scatter(didx[cur].at[jj + u], u)
                    start_gather(sidx[cur].at[jj + NBUF + u], u)
            elif blk + 1 < NBLK:
                pltpu.make_async_copy(src_hbm.at[wid, blk + 1], sidx[nxt],
                                      sem_i[nxt]).wait()
                pltpu.make_async_copy(dst_hbm.at[wid, blk + 1], didx[nxt],
                                      sem_i[nxt]).wait()
                for u in range(NBUF):
                    wait_scatter(didx[cur].at[jj + u], u)
                    start_gather(sidx[nxt].at[u], u)
            else:
                for u in range(NBUF):
                    wait_scatter(didx[cur].at[jj + u], u)

    plsc.subcore_barrier()

    for r0 in range(0, ROWS_PER_TILE, CH):
        a0 = sid * ROWS_PER_TILE + r0
        pltpu.sync_copy(acc_sh.at[pl.ds(a0, CH)], out_hbm.at[cid, pl.ds(a0, CH)])


def _dense_layer_body(p_ref, w_ref, b_ref, o_ref, *, activate):
    x = p_ref[0] + p_ref[1]
    y = jnp.dot(x, w_ref[...], preferred_element_type=jnp.float32) + b_ref[...]
    if activate:
        y = jnp.tanh(y)
    o_ref[...] = y


def _dense_layer(p, wt, b, activate):
    blk = 1024
    return pl.pallas_call(
        functools.partial(_dense_layer_body, activate=activate),
        grid=(NP // blk,),
        in_specs=[
            pl.BlockSpec((NUM_CORES, blk, D), lambda i: (0, i, 0)),
            pl.BlockSpec((D, D), lambda i: (0, 0)),
            pl.BlockSpec((1, D), lambda i: (0, 0)),
        ],
        out_specs=pl.BlockSpec((blk, D), lambda i: (i, 0)),
        out_shape=jax.ShapeDtypeStruct((NP, D), jnp.float32),
    )(p, wt, b)


@jax.jit
def kernel(feat, edge_index, W1, b1, W2, b2):
    src = edge_index[0]
    dst = edge_index[1]
    npad = EPAD - E
    pad_src = jnp.arange(npad, dtype=jnp.int32) % N
    pad_dst = N + jnp.arange(npad, dtype=jnp.int32) % (NP - N)
    src4 = jnp.concatenate([src, pad_src]).reshape(NW, NBLK, B, CH)
    dst4 = jnp.concatenate([dst, pad_dst]).reshape(NW, NBLK, B, CH)
    feat_p = jnp.pad(feat, ((0, NP - N), (0, 0)))

    p1 = _sc_segment_sum(feat_p, src4, dst4)
    h = _dense_layer(p1, W1.T, b1.reshape(1, D), activate=True)
    p2 = _sc_segment_sum(h, src4, dst4)
    out = _dense_layer(p2, W2.T, b2.reshape(1, D), activate=False)
    return out[:N]
